# baseline (speedup 1.0000x reference)
import functools

import jax
import jax.numpy as jnp
from jax import lax
from jax.experimental import pallas as pl
from jax.experimental.pallas import tpu as pltpu
from jax.experimental.pallas import tpu_sc as plsc

_D = 64
_SCALE = _D ** -0.5
_NBUF = 4
_L = 16


@functools.lru_cache(maxsize=None)
def _build(n_seq: int, seq_len: int, vocab: int):
    info = plsc.get_sparse_core_info()
    nw = info.num_cores * info.num_subcores
    spw = n_seq // nw
    assert n_seq % nw == 0 and spw % 128 == 0 and _D % 8 == 0

    mesh = plsc.VectorSubcoreMesh(core_axis_name="c", subcore_axis_name="s")

    scratch = (
        [pltpu.VMEM((seq_len, spw), jnp.int32)]
        + [pltpu.VMEM((spw, _D), jnp.float32) for _ in range(_NBUF)]
        + [pltpu.VMEM((_D // 8, 1, 8, spw), jnp.float32) for _ in range(_NBUF)]
        + [pltpu.SemaphoreType.DMA for _ in range(2 * _NBUF + 1)]
    )

    @functools.partial(
        pl.kernel,
        out_type=jax.ShapeDtypeStruct(
            (seq_len, _D // 8, n_seq // spw, 8, spw), jnp.float32
        ),
        mesh=mesh,
        scratch_types=scratch,
        compiler_params=pltpu.CompilerParams(
            use_tc_tiling_on_sc=False, needs_layout_passes=False
        ),
    )
    def emb_kernel(table_hbm, xt_hbm, out_hbm, *sc):
        idx_v = sc[0]
        gbufs = sc[1 : 1 + _NBUF]
        tbufs = sc[1 + _NBUF : 1 + 2 * _NBUF]
        gsems = sc[1 + 2 * _NBUF : 1 + 3 * _NBUF]
        osems = sc[1 + 3 * _NBUF : 1 + 4 * _NBUF]
        isem = sc[1 + 4 * _NBUF]

        wid = lax.axis_index("s") * info.num_cores + lax.axis_index("c")

        pltpu.async_copy(
            xt_hbm.at[:, pl.ds(wid * spw, spw)], idx_v, isem
        ).wait()

        def start_gather(p, b):
            pltpu.async_copy(
                table_hbm.at[idx_v.at[p]], gbufs[b], gsems[b]
            )

        def wait_gather(b):
            pltpu.make_async_copy(
                table_hbm.at[idx_v.at[0]], gbufs[b], gsems[b]
            ).wait()

        def start_out(p, b):
            pltpu.async_copy(
                tbufs[b],
                out_hbm.at[p, pl.ds(0, _D // 8), pl.ds(wid, 1)],
                osems[b],
            )

        def wait_out(b):
            pltpu.make_async_copy(
                tbufs[b],
                out_hbm.at[0, pl.ds(0, _D // 8), pl.ds(0, 1)],
                osems[b],
            ).wait()

        for b in range(_NBUF):
            start_gather(b, b)

        lanes = lax.iota(jnp.int32, _L)

        def round_body(t, carry):
            for b in range(_NBUF):
                p = t * _NBUF + b
                wait_gather(b)

                @pl.when(t > 0)
                def _():
                    wait_out(b)

                def col_body(dt, _):
                    for dr in range(8):
                        d = dt * 8 + dr
                        for j in range(spw // _L):
                            v = plsc.load_gather(
                                gbufs[b],
                                [lanes + j * _L, jnp.full((_L,), d, jnp.int32)],
                            )
                            tbufs[b][dt, 0, dr, pl.ds(j * _L, _L)] = v * _SCALE
                    return 0

                lax.fori_loop(0, _D // 8, col_body, 0)

                @pl.when(p + _NBUF < seq_len)
                def _():
                    start_gather(p + _NBUF, b)

                start_out(p, b)
            return carry

        lax.fori_loop(0, seq_len // _NBUF, round_body, 0)

        for b in range(_NBUF):
            wait_out(b)

    return emb_kernel


def kernel(x, table):
    n_seq, seq_len = x.shape
    out5 = _build(n_seq, seq_len, table.shape[0])(
        table, x.astype(jnp.int32).T
    )
    return out5.transpose(2, 4, 0, 1, 3).reshape(n_seq, seq_len, _D)


# speedup vs baseline: 1.9800x; 1.9800x over previous
import functools

import jax
import jax.numpy as jnp
from jax import lax
from jax.experimental import pallas as pl
from jax.experimental.pallas import tpu as pltpu
from jax.experimental.pallas import tpu_sc as plsc

_D = 64
_DP = 128
_SCALE = _D ** -0.5
_CHUNK = 64
_NBUF = 8
_LOOK = _NBUF // 2


@functools.lru_cache(maxsize=None)
def _build(n_idx: int, vocab: int):
    info = plsc.get_sparse_core_info()
    nw = info.num_cores * info.num_subcores
    per_w = n_idx // nw
    assert n_idx % nw == 0 and per_w % _CHUNK == 0
    n_chunks = per_w // _CHUNK

    mesh = plsc.VectorSubcoreMesh(core_axis_name="c", subcore_axis_name="s")

    scratch = (
        [pltpu.VMEM((per_w,), jnp.int32)]
        + [pltpu.VMEM((_CHUNK, _DP), jnp.float32) for _ in range(_NBUF)]
        + [pltpu.SemaphoreType.DMA for _ in range(2 * _NBUF + 1)]
    )

    @functools.partial(
        pl.kernel,
        out_type=jax.ShapeDtypeStruct((n_idx // _CHUNK, _CHUNK, _DP), jnp.float32),
        mesh=mesh,
        scratch_types=scratch,
        compiler_params=pltpu.CompilerParams(use_tc_tiling_on_sc=True),
    )
    def emb_kernel(table_hbm, x_hbm, out_hbm, *sc):
        idx_v = sc[0]
        gbufs = sc[1 : 1 + _NBUF]
        gsems = sc[1 + _NBUF : 1 + 2 * _NBUF]
        osems = sc[1 + 2 * _NBUF : 1 + 3 * _NBUF]
        isem = sc[1 + 3 * _NBUF]

        wid = lax.axis_index("s") * info.num_cores + lax.axis_index("c")
        base = wid * per_w

        pltpu.async_copy(x_hbm.at[pl.ds(base, per_w)], idx_v, isem).wait()

        def start_gather(c, b):
            pltpu.async_copy(
                table_hbm.at[idx_v.at[pl.ds(c * _CHUNK, _CHUNK)]],
                gbufs[b],
                gsems[b],
            )

        def wait_gather(b):
            pltpu.make_async_copy(
                table_hbm.at[idx_v.at[pl.ds(0, _CHUNK)]], gbufs[b], gsems[b]
            ).wait()

        def start_out(c, b):
            pltpu.async_copy(
                gbufs[b], out_hbm.at[base // _CHUNK + c], osems[b]
            )

        def wait_out(b):
            pltpu.make_async_copy(
                gbufs[b], out_hbm.at[0], osems[b]
            ).wait()

        for b in range(_LOOK):
            start_gather(b, b)

        def round_body(t, carry):
            for b in range(_NBUF):
                c = t * _NBUF + b
                f = (b + _LOOK) % _NBUF
                wait_gather(b)
                start_out(c, b)

                @pl.when(c + _LOOK < n_chunks)
                def _():
                    @pl.when(c >= _LOOK)
                    def _():
                        wait_out(f)

                    start_gather(c + _LOOK, f)
            return carry

        lax.fori_loop(0, n_chunks // _NBUF, round_body, 0)

        for b in range(_NBUF - _LOOK, _NBUF):
            wait_out(b)
        for b in range(_LOOK):
            wait_out(b)

    return emb_kernel


def kernel(x, table):
    n_idx = x.shape[0] * x.shape[1]
    tp = jnp.pad(table, ((0, 0), (0, _DP - _D))) * _SCALE
    xflat = x.astype(jnp.int32).reshape(n_idx)
    out3 = _build(n_idx, table.shape[0])(tp, xflat)
    out = out3.reshape(n_idx, _DP)[:, :_D]
    return out.reshape(x.shape[0], x.shape[1], _D)
